# Initial kernel scaffold; baseline (speedup 1.0000x reference)
import jax
import jax.numpy as jnp
from jax.experimental import pallas as pl


def kernel(x, edge_index, W1, b1, W2, b2, W3, b3, Wl, bl):
    raise NotImplementedError("write your pallas kernel here")


# speedup vs baseline: 21.6717x; 21.6717x over previous
import functools

import jax
import jax.numpy as jnp
from jax import lax
from jax.experimental import pallas as pl
from jax.experimental.pallas import tpu as pltpu
from jax.experimental.pallas import tpu_sc as plsc

N = 10000
NP = 10240
H = 128
NC = 2
NS = 16
NW = NC * NS
CH = 128
ROWS = NP // NS


def _mesh():
    return plsc.VectorSubcoreMesh(core_axis_name="c", subcore_axis_name="s")


G = 16


def _sc_edge(hp, srcm, dstm, zeros, nchunk):
    @functools.partial(
        pl.kernel,
        out_type=jax.ShapeDtypeStruct((NC, NP, H), jnp.float32),
        mesh=_mesh(),
        scratch_types=[
            pltpu.VMEM((G, CH), jnp.int32),
            pltpu.VMEM((G, CH), jnp.int32),
            pltpu.VMEM((CH, H), jnp.float32),
            pltpu.VMEM((CH, H), jnp.float32),
            pltpu.VMEM_SHARED((NP, H), jnp.float32),
            pltpu.SemaphoreType.DMA,
            pltpu.SemaphoreType.DMA,
        ],
    )
    def k(hp_hbm, srcm_hbm, dstm_hbm, z_hbm, out_hbm,
          idxs, idxd, buf0, buf1, acc_sh, gsem0, gsem1):
        c = lax.axis_index("c")
        s = lax.axis_index("s")
        wid = c * NS + s
        pltpu.sync_copy(z_hbm.at[pl.ds(s * ROWS, ROWS)],
                        acc_sh.at[pl.ds(s * ROWS, ROWS)])
        plsc.subcore_barrier()

        def group(g, carry):
            pltpu.sync_copy(srcm_hbm.at[wid, pl.ds(g * G, G)], idxs)
            pltpu.sync_copy(dstm_hbm.at[wid, pl.ds(g * G, G)], idxd)
            pltpu.async_copy(hp_hbm.at[idxs.at[0]], buf0, gsem0)

            def body(j, carry2):
                i = j * 2
                pltpu.async_copy(hp_hbm.at[idxs.at[i + 1]], buf1, gsem1)
                pltpu.make_async_copy(
                    hp_hbm.at[idxs.at[i]], buf0, gsem0).wait()
                pltpu.sync_copy(buf0, acc_sh.at[idxd.at[i]], add=True)

                @pl.when(i + 2 < G)
                def _():
                    pltpu.async_copy(hp_hbm.at[idxs.at[i + 2]], buf0, gsem0)

                pltpu.make_async_copy(
                    hp_hbm.at[idxs.at[i + 1]], buf1, gsem1).wait()
                pltpu.sync_copy(buf1, acc_sh.at[idxd.at[i + 1]], add=True)
                return carry2

            lax.fori_loop(0, G // 2, body, 0)
            return carry

        lax.fori_loop(0, nchunk // G, group, 0)
        plsc.subcore_barrier()
        pltpu.sync_copy(acc_sh.at[pl.ds(s * ROWS, ROWS)],
                        out_hbm.at[c, pl.ds(s * ROWS, ROWS)])

    return k(hp, srcm, dstm, zeros)


def _tc_pre_body(x_ref, w_ref, dp_ref, hp_ref, dinv_ref):
    dp = dp_ref[...]
    deg = dp[0, :, 0:1] + dp[1, :, 0:1] + 1.0
    dinv = lax.rsqrt(deg)
    h = jnp.dot(x_ref[...], w_ref[...], preferred_element_type=jnp.float32)
    hp_ref[...] = h * dinv
    dinv_ref[...] = dinv


def _tc_mid_body(p_ref, hp_ref, dinv_ref, b_ref, w_ref, out_ref):
    p = p_ref[...]
    acc = p[0] + p[1] + hp_ref[...]
    h = dinv_ref[...] * acc + b_ref[...]
    h = jnp.maximum(h, 0.0)
    out_ref[...] = dinv_ref[...] * jnp.dot(
        h, w_ref[...], preferred_element_type=jnp.float32)


def _tc_fin_body(p_ref, hp_ref, dinv_ref, b_ref, wl_ref, bl_ref, out_ref):
    p = p_ref[...]
    acc = p[0] + p[1] + hp_ref[...]
    h = dinv_ref[...] * acc + b_ref[...]
    out_ref[...] = jnp.dot(
        h, wl_ref[...], preferred_element_type=jnp.float32) + bl_ref[...]


def kernel(x, edge_index, W1, b1, W2, b2, W3, b3, Wl, bl):
    ei = edge_index.astype(jnp.int32)
    e = ei.shape[1]
    nchunk = -(-e // (NW * CH))
    nchunk = -(-nchunk // G) * G
    ep = NW * CH * nchunk
    pad_idx = N + jnp.arange(ep - e, dtype=jnp.int32) % (NP - N)
    src = jnp.concatenate([ei[0], pad_idx]).reshape(NW, nchunk, CH)
    dst = jnp.concatenate([ei[1], pad_idx]).reshape(NW, nchunk, CH)
    xp = jnp.pad(x, ((0, NP - N), (0, 0)))
    zeros = jnp.zeros((NP, H), jnp.float32)
    ones = jnp.ones((NP, H), jnp.float32)
    nout = Wl.shape[0]

    degp = _sc_edge(ones, src, dst, zeros, nchunk)
    hp1, dinv = pl.pallas_call(
        _tc_pre_body,
        out_shape=[jax.ShapeDtypeStruct((NP, H), jnp.float32),
                   jax.ShapeDtypeStruct((NP, 1), jnp.float32)],
    )(xp, W1.T, degp)

    p1 = _sc_edge(hp1, src, dst, zeros, nchunk)
    hp2 = pl.pallas_call(
        _tc_mid_body,
        out_shape=jax.ShapeDtypeStruct((NP, H), jnp.float32),
    )(p1, hp1, dinv, b1.reshape(1, H), W2.T)

    p2 = _sc_edge(hp2, src, dst, zeros, nchunk)
    hp3 = pl.pallas_call(
        _tc_mid_body,
        out_shape=jax.ShapeDtypeStruct((NP, H), jnp.float32),
    )(p2, hp2, dinv, b2.reshape(1, H), W3.T)

    p3 = _sc_edge(hp3, src, dst, zeros, nchunk)
    outp = pl.pallas_call(
        _tc_fin_body,
        out_shape=jax.ShapeDtypeStruct((NP, nout), jnp.float32),
    )(p3, hp3, dinv, b3.reshape(1, H), Wl.T, bl.reshape(1, nout))
    return outp[:N]


# speedup vs baseline: 28.0334x; 1.2935x over previous
import functools

import jax
import jax.numpy as jnp
from jax import lax
from jax.experimental import pallas as pl
from jax.experimental.pallas import tpu as pltpu
from jax.experimental.pallas import tpu_sc as plsc

N = 10000
NP = 10240
H = 128
NC = 2
NS = 16
NW = NC * NS
CH = 128
ROWS = NP // NS


def _mesh():
    return plsc.VectorSubcoreMesh(core_axis_name="c", subcore_axis_name="s")


G = 40


def _sc_degree(dstm, zeros1, ones1, nchunk):
    @functools.partial(
        pl.kernel,
        out_type=jax.ShapeDtypeStruct((NC, NP), jnp.float32),
        mesh=_mesh(),
        scratch_types=[
            pltpu.VMEM((nchunk, CH), jnp.int32),
            pltpu.VMEM((CH,), jnp.float32),
            pltpu.VMEM_SHARED((NP,), jnp.float32),
        ],
    )
    def k(dstm_hbm, z_hbm, ones_hbm, out_hbm, idxd, ones_v, deg_sh):
        c = lax.axis_index("c")
        s = lax.axis_index("s")
        wid = c * NS + s
        rows1 = NP // NS
        pltpu.sync_copy(z_hbm.at[pl.ds(s * rows1, rows1)],
                        deg_sh.at[pl.ds(s * rows1, rows1)])
        pltpu.sync_copy(dstm_hbm.at[wid], idxd)
        pltpu.sync_copy(ones_hbm, ones_v)
        plsc.subcore_barrier()

        def body(i, carry):
            pltpu.sync_copy(ones_v, deg_sh.at[idxd.at[i]], add=True)
            return carry

        lax.fori_loop(0, nchunk, body, 0)
        plsc.subcore_barrier()
        pltpu.sync_copy(deg_sh.at[pl.ds(s * rows1, rows1)],
                        out_hbm.at[c, pl.ds(s * rows1, rows1)])

    return k(dstm, zeros1, ones1)


def _sc_edge(hp, srcm, dstm, zeros, nchunk):
    @functools.partial(
        pl.kernel,
        out_type=jax.ShapeDtypeStruct((NC, NP, H), jnp.float32),
        mesh=_mesh(),
        scratch_types=[
            pltpu.VMEM((G, CH), jnp.int32),
            pltpu.VMEM((G, CH), jnp.int32),
            pltpu.VMEM((CH, H), jnp.float32),
            pltpu.VMEM((CH, H), jnp.float32),
            pltpu.VMEM_SHARED((NP, H), jnp.float32),
            pltpu.SemaphoreType.DMA,
            pltpu.SemaphoreType.DMA,
        ],
    )
    def k(hp_hbm, srcm_hbm, dstm_hbm, z_hbm, out_hbm,
          idxs, idxd, buf0, buf1, acc_sh, gsem0, gsem1):
        c = lax.axis_index("c")
        s = lax.axis_index("s")
        wid = c * NS + s
        pltpu.sync_copy(z_hbm.at[pl.ds(s * ROWS, ROWS)],
                        acc_sh.at[pl.ds(s * ROWS, ROWS)])
        plsc.subcore_barrier()

        def group(g, carry):
            pltpu.sync_copy(srcm_hbm.at[wid, pl.ds(g * G, G)], idxs)
            pltpu.sync_copy(dstm_hbm.at[wid, pl.ds(g * G, G)], idxd)
            pltpu.async_copy(hp_hbm.at[idxs.at[0]], buf0, gsem0)

            def body(j, carry2):
                i = j * 2
                pltpu.async_copy(hp_hbm.at[idxs.at[i + 1]], buf1, gsem1)
                pltpu.make_async_copy(
                    hp_hbm.at[idxs.at[i]], buf0, gsem0).wait()
                pltpu.sync_copy(buf0, acc_sh.at[idxd.at[i]], add=True)

                @pl.when(i + 2 < G)
                def _():
                    pltpu.async_copy(hp_hbm.at[idxs.at[i + 2]], buf0, gsem0)

                pltpu.make_async_copy(
                    hp_hbm.at[idxs.at[i + 1]], buf1, gsem1).wait()
                pltpu.sync_copy(buf1, acc_sh.at[idxd.at[i + 1]], add=True)
                return carry2

            lax.fori_loop(0, G // 2, body, 0)
            return carry

        lax.fori_loop(0, nchunk // G, group, 0)
        plsc.subcore_barrier()
        pltpu.sync_copy(acc_sh.at[pl.ds(s * ROWS, ROWS)],
                        out_hbm.at[c, pl.ds(s * ROWS, ROWS)])

    return k(hp, srcm, dstm, zeros)


def _tc_pre_body(x_ref, w_ref, dp_ref, hp_ref, dinv_ref):
    dp = dp_ref[...]
    deg = dp[:, 0:1] + dp[:, 1:2] + 1.0
    dinv = lax.rsqrt(deg)
    h = jnp.dot(x_ref[...], w_ref[...], preferred_element_type=jnp.float32)
    hp_ref[...] = h * dinv
    dinv_ref[...] = dinv


def _tc_mid_body(p_ref, hp_ref, dinv_ref, b_ref, w_ref, out_ref):
    p = p_ref[...]
    acc = p[0] + p[1] + hp_ref[...]
    h = dinv_ref[...] * acc + b_ref[...]
    h = jnp.maximum(h, 0.0)
    out_ref[...] = dinv_ref[...] * jnp.dot(
        h, w_ref[...], preferred_element_type=jnp.float32)


def _tc_fin_body(p_ref, hp_ref, dinv_ref, b_ref, wl_ref, bl_ref, out_ref):
    p = p_ref[...]
    acc = p[0] + p[1] + hp_ref[...]
    h = dinv_ref[...] * acc + b_ref[...]
    out_ref[...] = jnp.dot(
        h, wl_ref[...], preferred_element_type=jnp.float32) + bl_ref[...]


def kernel(x, edge_index, W1, b1, W2, b2, W3, b3, Wl, bl):
    ei = edge_index.astype(jnp.int32)
    e = ei.shape[1]
    nchunk = -(-e // (NW * CH))
    nchunk = -(-nchunk // G) * G
    ep = NW * CH * nchunk
    pad_idx = N + jnp.arange(ep - e, dtype=jnp.int32) % (NP - N)
    src = jnp.concatenate([ei[0], pad_idx]).reshape(NW, nchunk, CH)
    dst = jnp.concatenate([ei[1], pad_idx]).reshape(NW, nchunk, CH)
    xp = jnp.pad(x, ((0, NP - N), (0, 0)))
    zeros = jnp.zeros((NP, H), jnp.float32)
    nout = Wl.shape[0]

    degp = _sc_degree(dst, jnp.zeros((NP,), jnp.float32),
                      jnp.ones((CH,), jnp.float32), nchunk).T
    hp1, dinv = pl.pallas_call(
        _tc_pre_body,
        out_shape=[jax.ShapeDtypeStruct((NP, H), jnp.float32),
                   jax.ShapeDtypeStruct((NP, 1), jnp.float32)],
    )(xp, W1.T, degp)

    p1 = _sc_edge(hp1, src, dst, zeros, nchunk)
    hp2 = pl.pallas_call(
        _tc_mid_body,
        out_shape=jax.ShapeDtypeStruct((NP, H), jnp.float32),
    )(p1, hp1, dinv, b1.reshape(1, H), W2.T)

    p2 = _sc_edge(hp2, src, dst, zeros, nchunk)
    hp3 = pl.pallas_call(
        _tc_mid_body,
        out_shape=jax.ShapeDtypeStruct((NP, H), jnp.float32),
    )(p2, hp2, dinv, b2.reshape(1, H), W3.T)

    p3 = _sc_edge(hp3, src, dst, zeros, nchunk)
    outp = pl.pallas_call(
        _tc_fin_body,
        out_shape=jax.ShapeDtypeStruct((NP, nout), jnp.float32),
    )(p3, hp3, dinv, b3.reshape(1, H), Wl.T, bl.reshape(1, nout))
    return outp[:N]
